# baseline (speedup 1.0000x reference)
import jax
import jax.numpy as jnp
from jax import lax
from jax.experimental import pallas as pl
from jax.experimental.pallas import tpu as pltpu
from jax.experimental.pallas import tpu_sc as plsc

_D = 512


def _body(l1_hbm, l0_hbm, out_hbm):
    wid = lax.axis_index("s") * 2 + lax.axis_index("c")

    @pl.when(wid < 16)
    def _():
        pltpu.sync_copy(l1_hbm.at[pl.ds(wid * 4, 4)], out_hbm.at[pl.ds(wid * 4, 4)])

    @pl.when(wid >= 16)
    def _():
        m = wid - 16
        pltpu.sync_copy(
            l0_hbm.at[pl.ds(m * 4, 4), 0], out_hbm.at[pl.ds(64 + m * 4, 4)]
        )


@jax.jit
def kernel(layer1, layer0):
    mesh = plsc.VectorSubcoreMesh(core_axis_name="c", subcore_axis_name="s")
    f = pl.kernel(
        _body,
        out_type=jax.ShapeDtypeStruct((128, 2, _D), jnp.float32),
        mesh=mesh,
    )
    l1_p = layer1.reshape(8192, 2, _D)
    l0_q = layer0.reshape(4096, 2, 2, _D)
    return f(l1_p, l0_q).reshape(256, _D)


# speedup vs baseline: 1.1682x; 1.1682x over previous
import jax
import jax.numpy as jnp
from jax.experimental import pallas as pl
from jax.experimental.pallas import tpu as pltpu
from jax.experimental.pallas import tpu_sc as plsc

_D = 512


def _body(l1_hbm, l0_hbm, out_hbm, buf, sem1, sem0):
    c1 = pltpu.make_async_copy(l1_hbm.at[pl.ds(0, 64)], buf.at[pl.ds(0, 64)], sem1)
    c0 = pltpu.make_async_copy(
        l0_hbm.at[pl.ds(0, 64), 0], buf.at[pl.ds(64, 64)], sem0
    )
    c1.start()
    c0.start()
    c1.wait()
    c0.wait()
    pltpu.sync_copy(buf, out_hbm)


@jax.jit
def kernel(layer1, layer0):
    mesh = plsc.ScalarSubcoreMesh(axis_name="c", num_cores=1)
    f = pl.kernel(
        _body,
        out_type=jax.ShapeDtypeStruct((128, 2, _D), jnp.float32),
        mesh=mesh,
        scratch_types=[
            pltpu.VMEM_SHARED((128, 2, _D), jnp.float32),
            pltpu.SemaphoreType.DMA,
            pltpu.SemaphoreType.DMA,
        ],
    )
    l1_p = layer1.reshape(8192, 2, _D)
    l0_q = layer0.reshape(4096, 2, 2, _D)
    return f(l1_p, l0_q).reshape(256, _D)
